# Initial kernel scaffold; baseline (speedup 1.0000x reference)
import jax
import jax.numpy as jnp
from jax.experimental import pallas as pl


def kernel(x, a, b, theta):
    raise NotImplementedError("write your pallas kernel here")


# speedup vs baseline: 2.8303x; 2.8303x over previous
import functools

import jax
import jax.numpy as jnp
from jax import lax
from jax.experimental import pallas as pl
from jax.experimental.pallas import tpu as pltpu
from jax.experimental.pallas import tpu_sc as plsc

_NUM_OPT = 4
_NUM_D = 16
_LANES = 16
_NC = 2
_NS = 16
_NW = _NC * _NS
_BATCH = 16384
_RPW = _BATCH // _NW
_CHUNK = 128
_NCHUNK = _RPW // _CHUNK
_AW = _NUM_OPT * _NUM_D


def _irt_body(uids_hbm, qids_hbm, qids4_hbm, theta_hbm, a_hbm, b4_hbm, out_hbm,
              uid_v, qid_v, qid4_v, th_v, a_v, b4_v, o_v, sem):
    wid = lax.axis_index("s") * _NC + lax.axis_index("c")

    pltpu.sync_copy(uids_hbm.at[wid], uid_v)
    pltpu.sync_copy(qids_hbm.at[wid], qid_v)
    pltpu.sync_copy(qids4_hbm.at[wid], qid4_v)

    copies = []
    for j in range(_NCHUNK):
        sl = pl.ds(j * _CHUNK, _CHUNK)
        copies.append(pltpu.async_copy(theta_hbm.at[uid_v.at[j]], th_v.at[sl], sem))
        copies.append(pltpu.async_copy(a_hbm.at[qid_v.at[j]], a_v.at[sl], sem))
        copies.append(pltpu.async_copy(b4_hbm.at[qid4_v.at[j]], b4_v.at[sl], sem))
    for c in copies:
        c.wait()

    lanes = lax.iota(jnp.int32, _LANES)

    def blk_body(blk, carry):
        rows = lanes + blk * _LANES
        rhi = lax.shift_right_logical(rows, 7)
        rlo = lax.bitwise_and(rows, 127)
        qv = plsc.load_gather(qid_v, [rhi, rlo])
        bcol = lax.shift_left(lax.bitwise_and(qv, 3), 2)
        acc = [plsc.load_gather(b4_v, [rows, bcol + k]) for k in range(_NUM_OPT)]
        for d in range(_NUM_D):
            th_d = plsc.load_gather(th_v, [rows, jnp.full((_LANES,), d, jnp.int32)])
            for k in range(_NUM_OPT):
                a_kd = plsc.load_gather(
                    a_v, [rows, jnp.full((_LANES,), k * _NUM_D + d, jnp.int32)])
                acc[k] = acc[k] + a_kd * th_d
        for k in range(_NUM_OPT):
            flat = rows * _NUM_OPT + k
            plsc.store_scatter(
                o_v,
                [lax.shift_right_logical(flat, 7), lax.bitwise_and(flat, 127)],
                acc[k])
        return carry

    lax.fori_loop(0, _RPW // _LANES, blk_body, 0)

    nrow_o = _RPW * _NUM_OPT // 128
    pltpu.sync_copy(o_v, out_hbm.at[pl.ds(wid * nrow_o, nrow_o)])


_sc_call = functools.partial(
    pl.kernel,
    mesh=plsc.VectorSubcoreMesh(core_axis_name="c", subcore_axis_name="s"),
    compiler_params=pltpu.CompilerParams(
        needs_layout_passes=False, use_tc_tiling_on_sc=False),
    out_type=jax.ShapeDtypeStruct((_BATCH * _NUM_OPT // 128, 128), jnp.float32),
    scratch_types=[
        pltpu.VMEM((_NCHUNK, _CHUNK), jnp.int32),
        pltpu.VMEM((_NCHUNK, _CHUNK), jnp.int32),
        pltpu.VMEM((_NCHUNK, _CHUNK), jnp.int32),
        pltpu.VMEM((_RPW, _NUM_D), jnp.float32),
        pltpu.VMEM((_RPW, _AW), jnp.float32),
        pltpu.VMEM((_RPW, _NUM_D), jnp.float32),
        pltpu.VMEM((_RPW * _NUM_OPT // 128, 128), jnp.float32),
        pltpu.SemaphoreType.DMA,
    ],
)(_irt_body)


@jax.jit
def kernel(x, a, b, theta):
    uids = x[:, 0].astype(jnp.int32).reshape(_NW, _NCHUNK, _CHUNK)
    qids = x[:, 1].astype(jnp.int32).reshape(_NW, _NCHUNK, _CHUNK)
    qids4 = lax.shift_right_logical(qids, 2)
    a2 = a.reshape(a.shape[0], _AW)
    b4 = b.reshape(b.shape[0] * _NUM_OPT // _NUM_D, _NUM_D)
    out = _sc_call(uids, qids, qids4, theta, a2, b4)
    return out.reshape(_BATCH, _NUM_OPT)
